# baseline (speedup 1.0000x reference)
import functools
import math

import jax
import jax.numpy as jnp
from jax import lax
from jax.experimental import pallas as pl
from jax.experimental.pallas import tpu as pltpu
from jax.experimental.pallas import tpu_sc as plsc

_NB = 4


@functools.lru_cache(maxsize=None)
def _make_lookup(b: int, s: int, vocab: int, d: int):
    info = plsc.get_sparse_core_info()
    nc, ns, nl = info.num_cores, info.num_subcores, info.num_lanes
    nw = nc * ns
    assert d % nl == 0
    assert b % (nw * _NB) == 0
    rows_per_w = b // nw
    n_chunks = rows_per_w // _NB
    scale = jnp.float32(math.sqrt(float(d)))
    mesh = plsc.VectorSubcoreMesh(core_axis_name="c", subcore_axis_name="s")

    @functools.partial(
        pl.kernel,
        mesh=mesh,
        out_type=jax.ShapeDtypeStruct((b, s, d), jnp.float32),
        scratch_types=[
            pltpu.VMEM((rows_per_w, s), jnp.int32),
            pltpu.VMEM((_NB, s, d), jnp.float32),
            pltpu.VMEM((_NB, s, d), jnp.float32),
            pltpu.SemaphoreType.DMA,
            pltpu.SemaphoreType.DMA,
        ],
    )
    def lookup(idx_hbm, table_hbm, out_hbm, idx_v, buf0, buf1, sem0, sem1):
        wid = lax.axis_index("s") * nc + lax.axis_index("c")
        base = wid * rows_per_w
        pltpu.sync_copy(idx_hbm.at[pl.ds(base, rows_per_w)], idx_v)

        def fire(g, buf, sem):
            for r in range(_NB):
                pltpu.async_copy(
                    table_hbm.at[idx_v.at[g * _NB + r]], buf.at[r], sem
                )

        def drain(buf, sem):
            pltpu.make_async_copy(out_hbm.at[pl.ds(0, _NB)], buf, sem).wait()

        def scale_buf(buf):
            @plsc.parallel_loop(0, s, 1, unroll=2)
            def _(i):
                for r in range(_NB):
                    for j in range(d // nl):
                        sl = pl.ds(j * nl, nl)
                        buf[r, i, sl] = buf[r, i, sl] * scale

        def emit(g, buf):
            pltpu.sync_copy(buf, out_hbm.at[pl.ds(base + g * _NB, _NB)])

        fire(0, buf0, sem0)

        def pair_body(h, carry):
            g0 = 2 * h
            fire(g0 + 1, buf1, sem1)
            drain(buf0, sem0)
            scale_buf(buf0)
            emit(g0, buf0)

            @pl.when(g0 + 2 < n_chunks)
            def _():
                fire(g0 + 2, buf0, sem0)

            drain(buf1, sem1)
            scale_buf(buf1)
            emit(g0 + 1, buf1)
            return carry

        lax.fori_loop(0, n_chunks // 2, pair_body, 0)

    return lookup


_SPLITS = 2


def kernel(inputs, table):
    b, s = inputs.shape
    vocab, d = table.shape
    idx = inputs.astype(jnp.int32)
    bp = b // _SPLITS
    lookup = _make_lookup(bp, s, vocab, d)
    parts = [lookup(idx[p * bp : (p + 1) * bp], table) for p in range(_SPLITS)]
    return jnp.concatenate(parts, axis=0)


# speedup vs baseline: 1.6121x; 1.6121x over previous
import functools
import math

import jax
import jax.numpy as jnp
from jax import lax
from jax.experimental import pallas as pl
from jax.experimental.pallas import tpu as pltpu
from jax.experimental.pallas import tpu_sc as plsc

_NB = 4


@functools.lru_cache(maxsize=None)
def _make_gather(bp: int, s: int, vocab: int, d: int):
    info = plsc.get_sparse_core_info()
    nc, ns, nl = info.num_cores, info.num_subcores, info.num_lanes
    nw = nc * ns
    assert d % nl == 0
    assert bp % (nw * _NB) == 0
    rows_per_w = bp // nw
    n_chunks = rows_per_w // _NB
    assert n_chunks % 2 == 0
    scale = jnp.float32(math.sqrt(float(d)))
    mesh = plsc.VectorSubcoreMesh(core_axis_name="c", subcore_axis_name="s")

    @functools.partial(
        pl.kernel,
        mesh=mesh,
        out_type=jax.ShapeDtypeStruct((bp, s, d), jnp.float32),
        scratch_types=[
            pltpu.VMEM((rows_per_w, s), jnp.int32),
            pltpu.VMEM((_NB, s, d), jnp.float32),
            pltpu.VMEM((_NB, s, d), jnp.float32),
            pltpu.SemaphoreType.DMA,
            pltpu.SemaphoreType.DMA,
        ],
    )
    def gather(idx_hbm, table_hbm, out_hbm, idx_v, buf0, buf1, sem0, sem1):
        wid = lax.axis_index("s") * nc + lax.axis_index("c")
        base = wid * rows_per_w
        pltpu.sync_copy(idx_hbm.at[pl.ds(base, rows_per_w)], idx_v)

        def fire(g, buf, sem):
            for r in range(_NB):
                pltpu.async_copy(
                    table_hbm.at[idx_v.at[g * _NB + r]], buf.at[r], sem
                )

        def drain(buf, sem):
            pltpu.make_async_copy(table_hbm.at[pl.ds(0, _NB * s)], buf, sem).wait()

        def scale_buf(buf):
            @plsc.parallel_loop(0, s, 1, unroll=2)
            def _(i):
                for r in range(_NB):
                    for j in range(d // nl):
                        sl = pl.ds(j * nl, nl)
                        buf[r, i, sl] = buf[r, i, sl] * scale

        def emit(g, buf):
            for r in range(_NB):
                row = base + g * _NB + r
                pltpu.sync_copy(buf.at[r], out_hbm.at[row])

        fire(0, buf0, sem0)

        def pair_body(h, carry):
            g0 = 2 * h
            fire(g0 + 1, buf1, sem1)
            drain(buf0, sem0)
            scale_buf(buf0)
            emit(g0, buf0)

            @pl.when(g0 + 2 < n_chunks)
            def _():
                fire(g0 + 2, buf0, sem0)

            drain(buf1, sem1)
            scale_buf(buf1)
            emit(g0 + 1, buf1)
            return carry

        lax.fori_loop(0, n_chunks // 2, pair_body, 0)

    return gather


def kernel(inputs, table):
    b, s = inputs.shape
    vocab, d = table.shape
    idx = inputs.astype(jnp.int32)
    return _make_gather(b, s, vocab, d)(idx, table)
